# Initial kernel scaffold; baseline (speedup 1.0000x reference)
import jax
import jax.numpy as jnp
from jax.experimental import pallas as pl


def kernel(params, input_ids):
    raise NotImplementedError("write your pallas kernel here")


# speedup vs baseline: 1.3983x; 1.3983x over previous
import functools

import jax
import jax.numpy as jnp
from jax import lax
from jax.experimental import pallas as pl
from jax.experimental.pallas import tpu as pltpu
from jax.experimental.pallas import tpu_sc as plsc

SEQ = 2048
DIM = 768
HEADS = 12
HEAD_DIM = 64
FF = 4 * DIM
LAYERS = 2
SCHEDULE = [1.0 - i / (LAYERS - 1) * 0.8 for i in range(LAYERS)]
SCALE = HEAD_DIM ** (-0.5)
RBLK = 256
QBLK = 256
EPS = 1e-5


def _sc_embed_gather(table, ids):
    info = plsc.get_sparse_core_info()
    nc, ns = info.num_cores, info.num_subcores
    nw = nc * ns
    b_per_w = SEQ // nw
    mesh = plsc.VectorSubcoreMesh(core_axis_name="c", subcore_axis_name="s")

    @functools.partial(
        pl.kernel,
        mesh=mesh,
        out_type=jax.ShapeDtypeStruct((SEQ, DIM), jnp.float32),
        scratch_types=[
            pltpu.VMEM((b_per_w,), jnp.int32),
            pltpu.VMEM((b_per_w, DIM), jnp.float32),
            pltpu.SemaphoreType.DMA,
        ],
    )
    def gather(table_hbm, idx_hbm, out_hbm, idx_v, rows_v, sem):
        wid = lax.axis_index("s") * nc + lax.axis_index("c")
        base = wid * b_per_w
        pltpu.sync_copy(idx_hbm.at[pl.ds(base, b_per_w)], idx_v)
        pltpu.async_copy(table_hbm.at[idx_v], rows_v, sem).wait()
        pltpu.sync_copy(rows_v, out_hbm.at[pl.ds(base, b_per_w)])

    return gather(table, ids)


_embed_gather = _sc_embed_gather


def _layernorm(x, g, b):
    m = jnp.mean(x, axis=-1, keepdims=True)
    v = jnp.mean((x - m) * (x - m), axis=-1, keepdims=True)
    return (x - m) * lax.rsqrt(v + EPS) * g + b


def _gelu(x):
    return 0.5 * x * (1.0 + lax.erf(x * (2.0 ** -0.5)))


def _dot(a, b):
    return jnp.dot(a, b, preferred_element_type=jnp.float32)


def _in_proj(emb, pos, w, b):
    def body(emb_ref, pos_ref, w_ref, b_ref, o_ref):
        h = emb_ref[...] + pos_ref[...]
        o_ref[...] = _dot(h, w_ref[...]) + b_ref[...]

    return pl.pallas_call(
        body,
        grid=(SEQ // RBLK,),
        in_specs=[
            pl.BlockSpec((RBLK, DIM), lambda i: (i, 0)),
            pl.BlockSpec((RBLK, DIM), lambda i: (i, 0)),
            pl.BlockSpec((DIM, DIM), lambda i: (0, 0)),
            pl.BlockSpec((1, DIM), lambda i: (0, 0)),
        ],
        out_specs=pl.BlockSpec((RBLK, DIM), lambda i: (i, 0)),
        out_shape=jax.ShapeDtypeStruct((SEQ, DIM), jnp.float32),
    )(emb, pos, w, b)


def _ln_qkv(x, g, b, qw, qb, kw, kb, vw, vb):
    def body(x_ref, g_ref, b_ref, qw_ref, qb_ref, kw_ref, kb_ref, vw_ref,
             vb_ref, q_ref, k_ref, v_ref):
        h = _layernorm(x_ref[...], g_ref[...], b_ref[...])
        q_ref[...] = _dot(h, qw_ref[...]) + qb_ref[...]
        k_ref[...] = _dot(h, kw_ref[...]) + kb_ref[...]
        v_ref[...] = _dot(h, vw_ref[...]) + vb_ref[...]

    row = pl.BlockSpec((RBLK, DIM), lambda i: (i, 0))
    wsp = pl.BlockSpec((DIM, DIM), lambda i: (0, 0))
    bsp = pl.BlockSpec((1, DIM), lambda i: (0, 0))
    out = jax.ShapeDtypeStruct((SEQ, DIM), jnp.float32)
    return pl.pallas_call(
        body,
        grid=(SEQ // RBLK,),
        in_specs=[row, bsp, bsp, wsp, bsp, wsp, bsp, wsp, bsp],
        out_specs=[row, row, row],
        out_shape=[out, out, out],
    )(x, g, b, qw, qb, kw, kb, vw, vb)


def _head_slice(ref, h):
    return ref[:, h * HEAD_DIM:(h + 1) * HEAD_DIM]


def _attention_full(q, k, v):
    def body(q_ref, k_ref, v_ref, o_ref):
        outs = []
        for h in range(HEADS):
            s = lax.dot_general(_head_slice(q_ref, h), _head_slice(k_ref, h),
                                (((1,), (1,)), ((), ())),
                                preferred_element_type=jnp.float32) * SCALE
            m = jnp.max(s, axis=-1, keepdims=True)
            e = jnp.exp(s - m)
            w = e / jnp.sum(e, axis=-1, keepdims=True)
            outs.append(_dot(w, _head_slice(v_ref, h)))
        o_ref[...] = jnp.concatenate(outs, axis=1)

    return pl.pallas_call(
        body,
        grid=(SEQ // QBLK,),
        in_specs=[
            pl.BlockSpec((QBLK, DIM), lambda i: (i, 0)),
            pl.BlockSpec((SEQ, DIM), lambda i: (0, 0)),
            pl.BlockSpec((SEQ, DIM), lambda i: (0, 0)),
        ],
        out_specs=pl.BlockSpec((QBLK, DIM), lambda i: (i, 0)),
        out_shape=jax.ShapeDtypeStruct((SEQ, DIM), jnp.float32),
    )(q, k, v)


def _importance_and_threshold(q, k, keep):
    nqb = SEQ // QBLK

    def body(q_ref, k_ref, imp_ref, thr_ref):
        i = pl.program_id(0)

        @pl.when(i == 0)
        def _():
            imp_ref[...] = jnp.zeros_like(imp_ref)

        c = jnp.zeros((1, SEQ), jnp.float32)
        for h in range(HEADS):
            s = lax.dot_general(_head_slice(q_ref, h), _head_slice(k_ref, h),
                                (((1,), (1,)), ((), ())),
                                preferred_element_type=jnp.float32)
            c = c + jnp.sum(jnp.abs(s), axis=0, keepdims=True)
        imp_ref[...] += jnp.broadcast_to(c, (8, SEQ))

        @pl.when(i == nqb - 1)
        def _():
            bits = lax.bitcast_convert_type(imp_ref[...], jnp.int32)
            target = keep * 8

            def step(j, t):
                cand = t | (1 << (30 - j))
                cnt = jnp.sum((bits >= cand).astype(jnp.int32))
                return jnp.where(cnt >= target, cand, t)

            t = lax.fori_loop(0, 31, step, jnp.int32(0))
            tf = lax.bitcast_convert_type(t, jnp.float32)
            thr_ref[...] = jnp.full((8, 128), tf, jnp.float32)

    return pl.pallas_call(
        body,
        grid=(nqb,),
        in_specs=[
            pl.BlockSpec((QBLK, DIM), lambda i: (i, 0)),
            pl.BlockSpec((SEQ, DIM), lambda i: (0, 0)),
        ],
        out_specs=[
            pl.BlockSpec((8, SEQ), lambda i: (0, 0)),
            pl.BlockSpec((8, 128), lambda i: (0, 0)),
        ],
        out_shape=[
            jax.ShapeDtypeStruct((8, SEQ), jnp.float32),
            jax.ShapeDtypeStruct((8, 128), jnp.float32),
        ],
    )(q, k)


def _attention_masked(q, k, v, imp, thr):
    def body(q_ref, k_ref, v_ref, imp_ref, thr_ref, o_ref):
        t = thr_ref[0, 0]
        keep = imp_ref[0:1, :] >= t
        outs = []
        for h in range(HEADS):
            s = lax.dot_general(_head_slice(q_ref, h), _head_slice(k_ref, h),
                                (((1,), (1,)), ((), ())),
                                preferred_element_type=jnp.float32) * SCALE
            s = jnp.where(keep, s, -1e30)
            m = jnp.max(s, axis=-1, keepdims=True)
            e = jnp.exp(s - m)
            w = e / jnp.sum(e, axis=-1, keepdims=True)
            outs.append(_dot(w, _head_slice(v_ref, h)))
        o_ref[...] = jnp.concatenate(outs, axis=1)

    return pl.pallas_call(
        body,
        grid=(SEQ // QBLK,),
        in_specs=[
            pl.BlockSpec((QBLK, DIM), lambda i: (i, 0)),
            pl.BlockSpec((SEQ, DIM), lambda i: (0, 0)),
            pl.BlockSpec((SEQ, DIM), lambda i: (0, 0)),
            pl.BlockSpec((8, SEQ), lambda i: (0, 0)),
            pl.BlockSpec((8, 128), lambda i: (0, 0)),
        ],
        out_specs=pl.BlockSpec((QBLK, DIM), lambda i: (i, 0)),
        out_shape=jax.ShapeDtypeStruct((SEQ, DIM), jnp.float32),
    )(q, k, v, imp, thr)


def _out_proj_residual(x, attn, w, b):
    def body(x_ref, a_ref, w_ref, b_ref, o_ref):
        o_ref[...] = x_ref[...] + _dot(a_ref[...], w_ref[...]) + b_ref[...]

    return pl.pallas_call(
        body,
        grid=(SEQ // RBLK,),
        in_specs=[
            pl.BlockSpec((RBLK, DIM), lambda i: (i, 0)),
            pl.BlockSpec((RBLK, DIM), lambda i: (i, 0)),
            pl.BlockSpec((DIM, DIM), lambda i: (0, 0)),
            pl.BlockSpec((1, DIM), lambda i: (0, 0)),
        ],
        out_specs=pl.BlockSpec((RBLK, DIM), lambda i: (i, 0)),
        out_shape=jax.ShapeDtypeStruct((SEQ, DIM), jnp.float32),
    )(x, attn, w, b)


def _ffn_residual(x, g, b, w1, b1, w2, b2):
    def body(x_ref, g_ref, b_ref, w1_ref, b1_ref, w2_ref, b2_ref, o_ref):
        h = _layernorm(x_ref[...], g_ref[...], b_ref[...])
        f = _gelu(_dot(h, w1_ref[...]) + b1_ref[...])
        o_ref[...] = x_ref[...] + _dot(f, w2_ref[...]) + b2_ref[...]

    return pl.pallas_call(
        body,
        grid=(SEQ // RBLK,),
        in_specs=[
            pl.BlockSpec((RBLK, DIM), lambda i: (i, 0)),
            pl.BlockSpec((1, DIM), lambda i: (0, 0)),
            pl.BlockSpec((1, DIM), lambda i: (0, 0)),
            pl.BlockSpec((DIM, FF), lambda i: (0, 0)),
            pl.BlockSpec((1, FF), lambda i: (0, 0)),
            pl.BlockSpec((FF, DIM), lambda i: (0, 0)),
            pl.BlockSpec((1, DIM), lambda i: (0, 0)),
        ],
        out_specs=pl.BlockSpec((RBLK, DIM), lambda i: (i, 0)),
        out_shape=jax.ShapeDtypeStruct((SEQ, DIM), jnp.float32),
    )(x, g, b, w1, b1, w2, b2)


def _row(v):
    return v.reshape(1, -1)


def kernel(params, input_ids):
    ids = input_ids.reshape(-1).astype(jnp.int32)
    emb = _embed_gather(params['tok_emb'], ids)
    pos = params['pos_emb'][:SEQ]
    x = _in_proj(emb, pos, params['in_w'], _row(params['in_b']))

    for li in range(LAYERS):
        p = params['layers'][li]
        ratio = SCHEDULE[li]
        q, k, v = _ln_qkv(x, _row(p['ln1_g']), _row(p['ln1_b']),
                          p['q_w'], _row(p['q_b']),
                          p['k_w'], _row(p['k_b']),
                          p['v_w'], _row(p['v_b']))
        if ratio < 1.0:
            num_keep = max(1, int(ratio * SEQ))
            imp, thr = _importance_and_threshold(q, k, num_keep)
            attn = _attention_masked(q, k, v, imp, thr)
        else:
            attn = _attention_full(q, k, v)
        x = _out_proj_residual(x, attn, p['out_w'], _row(p['out_b']))
        x = _ffn_residual(x, _row(p['ln2_g']), _row(p['ln2_b']),
                          p['ff1_w'], _row(p['ff1_b']),
                          p['ff2_w'], _row(p['ff2_b']))

    return x.reshape(1, SEQ, DIM)


# speedup vs baseline: 1.5778x; 1.1283x over previous
import functools

import jax
import jax.numpy as jnp
from jax import lax
from jax.experimental import pallas as pl
from jax.experimental.pallas import tpu as pltpu
from jax.experimental.pallas import tpu_sc as plsc

SEQ = 2048
DIM = 768
HEADS = 12
HEAD_DIM = 64
FF = 4 * DIM
LAYERS = 2
SCHEDULE = [1.0 - i / (LAYERS - 1) * 0.8 for i in range(LAYERS)]
SCALE = HEAD_DIM ** (-0.5)
RBLK = 256
QBLK = 256
EPS = 1e-5


def _sc_embed_gather(table, ids):
    info = plsc.get_sparse_core_info()
    nc, ns = info.num_cores, info.num_subcores
    nw = nc * ns
    b_per_w = SEQ // nw
    mesh = plsc.VectorSubcoreMesh(core_axis_name="c", subcore_axis_name="s")

    @functools.partial(
        pl.kernel,
        mesh=mesh,
        out_type=jax.ShapeDtypeStruct((SEQ, DIM), jnp.float32),
        scratch_types=[
            pltpu.VMEM((b_per_w,), jnp.int32),
            pltpu.VMEM((b_per_w, DIM), jnp.float32),
            pltpu.SemaphoreType.DMA,
        ],
    )
    def gather(table_hbm, idx_hbm, out_hbm, idx_v, rows_v, sem):
        wid = lax.axis_index("s") * nc + lax.axis_index("c")
        base = wid * b_per_w
        pltpu.sync_copy(idx_hbm.at[pl.ds(base, b_per_w)], idx_v)
        pltpu.async_copy(table_hbm.at[idx_v], rows_v, sem).wait()
        pltpu.sync_copy(rows_v, out_hbm.at[pl.ds(base, b_per_w)])

    return gather(table, ids)


_embed_gather = _sc_embed_gather

PAD = 512


def _sc_build_gather(k, v, posflat):
    mesh = plsc.VectorSubcoreMesh(core_axis_name="c", subcore_axis_name="s")
    ppt = SEQ // 16

    @functools.partial(
        pl.kernel,
        mesh=mesh,
        out_type=[jax.ShapeDtypeStruct((PAD, DIM), jnp.float32),
                  jax.ShapeDtypeStruct((PAD, DIM), jnp.float32)],
        scratch_types=[
            pltpu.VMEM((ppt,), jnp.int32),
            pltpu.VMEM((ppt,), jnp.int32),
            pltpu.VMEM((PAD,), jnp.int32),
            pltpu.VMEM((16,), jnp.int32),
            pltpu.VMEM((16, DIM), jnp.float32),
            pltpu.VMEM((16, DIM), jnp.float32),
            pltpu.VMEM_SHARED((PAD,), jnp.int32),
            pltpu.SemaphoreType.DMA,
            pltpu.SemaphoreType.DMA,
        ],
    )
    def build_gather(k_hbm, v_hbm, pos_hbm, kc_hbm, vc_hbm,
                     pos_v, src_v, zeros_v, slot_v, krows, vrows, shared,
                     sem1, sem2):
        sid = lax.axis_index("s")
        cid = lax.axis_index("c")

        @pl.when(sid == 0)
        def _():
            for c in range(PAD // 16):
                zeros_v[pl.ds(c * 16, 16)] = jnp.zeros((16,), jnp.int32)
            pltpu.sync_copy(zeros_v, shared)

        base = sid * ppt
        pltpu.sync_copy(pos_hbm.at[pl.ds(base, ppt)], pos_v)
        for c in range(ppt // 16):
            pv = pos_v[pl.ds(c * 16, 16)]
            dead = pv >= PAD
            sv = base + c * 16 + lax.iota(jnp.int32, 16)
            pos_v[pl.ds(c * 16, 16)] = jnp.where(dead, jnp.int32(PAD - 1), pv)
            src_v[pl.ds(c * 16, 16)] = jnp.where(dead, jnp.int32(0), sv)
        plsc.subcore_barrier()
        pltpu.sync_copy(src_v, shared.at[pos_v], add=True)
        plsc.subcore_barrier()

        wid = cid * 16 + sid
        pltpu.sync_copy(shared.at[pl.ds(wid * 16, 16)], slot_v)
        c1 = pltpu.async_copy(k_hbm.at[slot_v], krows, sem1)
        c2 = pltpu.async_copy(v_hbm.at[slot_v], vrows, sem2)
        c1.wait()
        c2.wait()
        pltpu.sync_copy(krows, kc_hbm.at[pl.ds(wid * 16, 16)])
        pltpu.sync_copy(vrows, vc_hbm.at[pl.ds(wid * 16, 16)])

    return build_gather(k, v, posflat)


def _posmap(imp2, keep):
    def body(imp_ref, o_ref):
        bits = lax.bitcast_convert_type(imp_ref[...], jnp.int32)

        def step(j, t):
            cand = t | (1 << (30 - j))
            cnt = jnp.sum((bits >= cand).astype(jnp.int32))
            return jnp.where(cnt >= keep, cand, t)

        t = lax.fori_loop(0, 31, step, jnp.int32(0))
        kf = (bits >= t).astype(jnp.float32)
        i_ = lax.broadcasted_iota(jnp.int32, (128, 128), 0)
        j_ = lax.broadcasted_iota(jnp.int32, (128, 128), 1)
        inrow = _dot(kf, (i_ <= j_).astype(jnp.float32))
        a_ = lax.broadcasted_iota(jnp.int32, (16, 16), 0)
        b_ = lax.broadcasted_iota(jnp.int32, (16, 16), 1)
        prev = _dot((b_ < a_).astype(jnp.float32), inrow[:, 127:128])
        pos = (inrow + prev).astype(jnp.int32) - 1
        o_ref[...] = jnp.where(bits >= t, pos, jnp.int32(PAD))

    return pl.pallas_call(
        body,
        in_specs=[pl.BlockSpec((16, 128), lambda: (0, 0))],
        out_specs=pl.BlockSpec((16, 128), lambda: (0, 0)),
        out_shape=jax.ShapeDtypeStruct((16, 128), jnp.int32),
    )(imp2)


_build_gather_fn = _sc_build_gather


def _layernorm(x, g, b):
    m = jnp.mean(x, axis=-1, keepdims=True)
    v = jnp.mean((x - m) * (x - m), axis=-1, keepdims=True)
    return (x - m) * lax.rsqrt(v + EPS) * g + b


def _gelu(x):
    return 0.5 * x * (1.0 + lax.erf(x * (2.0 ** -0.5)))


def _dot(a, b):
    return jnp.dot(a, b, preferred_element_type=jnp.float32)


def _in_proj(emb, pos, w, b):
    def body(emb_ref, pos_ref, w_ref, b_ref, o_ref):
        h = emb_ref[...] + pos_ref[...]
        o_ref[...] = _dot(h, w_ref[...]) + b_ref[...]

    return pl.pallas_call(
        body,
        grid=(SEQ // RBLK,),
        in_specs=[
            pl.BlockSpec((RBLK, DIM), lambda i: (i, 0)),
            pl.BlockSpec((RBLK, DIM), lambda i: (i, 0)),
            pl.BlockSpec((DIM, DIM), lambda i: (0, 0)),
            pl.BlockSpec((1, DIM), lambda i: (0, 0)),
        ],
        out_specs=pl.BlockSpec((RBLK, DIM), lambda i: (i, 0)),
        out_shape=jax.ShapeDtypeStruct((SEQ, DIM), jnp.float32),
    )(emb, pos, w, b)


def _bf(x):
    return x.astype(jnp.bfloat16)


def _ln_qkv(x, g, b, qw, qb, kw, kb, vw, vb):
    def body(x_ref, g_ref, b_ref, qw_ref, qb_ref, kw_ref, kb_ref, vw_ref,
             vb_ref, q_ref, k_ref, v_ref):
        h = _layernorm(x_ref[...], g_ref[...], b_ref[...])
        q_ref[...] = (_dot(h, qw_ref[...]) + qb_ref[...]) * SCALE
        k_ref[...] = _dot(h, kw_ref[...]) + kb_ref[...]
        v_ref[...] = _dot(h, vw_ref[...]) + vb_ref[...]

    row = pl.BlockSpec((RBLK, DIM), lambda i: (i, 0))
    wsp = pl.BlockSpec((DIM, DIM), lambda i: (0, 0))
    bsp = pl.BlockSpec((1, DIM), lambda i: (0, 0))
    out = jax.ShapeDtypeStruct((SEQ, DIM), jnp.float32)
    return pl.pallas_call(
        body,
        grid=(SEQ // RBLK,),
        in_specs=[row, bsp, bsp, wsp, bsp, wsp, bsp, wsp, bsp],
        out_specs=[row, row, row],
        out_shape=[out, out, out],
    )(x, g, b, qw, qb, kw, kb, vw, vb)


def _head_slice(ref, h):
    return ref[:, h * HEAD_DIM:(h + 1) * HEAD_DIM]


def _attention_full(q, k, v):
    def body(q_ref, k_ref, v_ref, o_ref):
        outs = []
        for h in range(HEADS):
            s = lax.dot_general(_head_slice(q_ref, h), _head_slice(k_ref, h),
                                (((1,), (1,)), ((), ())),
                                preferred_element_type=jnp.float32)
            m = jnp.max(s, axis=-1, keepdims=True)
            e = jnp.exp(s - m)
            w = e * (1.0 / jnp.sum(e, axis=-1, keepdims=True))
            outs.append(_dot(w, _head_slice(v_ref, h)))
        o_ref[...] = jnp.concatenate(outs, axis=1)

    return pl.pallas_call(
        body,
        grid=(SEQ // QBLK,),
        in_specs=[
            pl.BlockSpec((QBLK, DIM), lambda i: (i, 0)),
            pl.BlockSpec((SEQ, DIM), lambda i: (0, 0)),
            pl.BlockSpec((SEQ, DIM), lambda i: (0, 0)),
        ],
        out_specs=pl.BlockSpec((QBLK, DIM), lambda i: (i, 0)),
        out_shape=jax.ShapeDtypeStruct((SEQ, DIM), jnp.float32),
    )(q, k, v)


def _importance_and_threshold(q, k, keep):
    nqb = SEQ // QBLK

    def body(q_ref, k_ref, imp_ref, thr_ref):
        i = pl.program_id(0)

        @pl.when(i == 0)
        def _():
            imp_ref[...] = jnp.zeros_like(imp_ref)

        c = jnp.zeros((1, SEQ), jnp.float32)
        for h in range(HEADS):
            s = lax.dot_general(_head_slice(q_ref, h), _head_slice(k_ref, h),
                                (((1,), (1,)), ((), ())),
                                preferred_element_type=jnp.float32)
            c = c + jnp.sum(jnp.abs(s), axis=0, keepdims=True)
        imp_ref[...] += jnp.broadcast_to(c, (8, SEQ))

        @pl.when(i == nqb - 1)
        def _():
            bits = lax.bitcast_convert_type(imp_ref[...], jnp.int32)
            target = keep * 8

            def step(j, t):
                cand = t | (1 << (30 - j))
                cnt = jnp.sum((bits >= cand).astype(jnp.int32))
                return jnp.where(cnt >= target, cand, t)

            t = lax.fori_loop(0, 31, step, jnp.int32(0))
            tf = lax.bitcast_convert_type(t, jnp.float32)
            thr_ref[...] = jnp.full((8, 128), tf, jnp.float32)

    return pl.pallas_call(
        body,
        grid=(nqb,),
        in_specs=[
            pl.BlockSpec((QBLK, DIM), lambda i: (i, 0)),
            pl.BlockSpec((SEQ, DIM), lambda i: (0, 0)),
        ],
        out_specs=[
            pl.BlockSpec((8, SEQ), lambda i: (0, 0)),
            pl.BlockSpec((8, 128), lambda i: (0, 0)),
        ],
        out_shape=[
            jax.ShapeDtypeStruct((8, SEQ), jnp.float32),
            jax.ShapeDtypeStruct((8, 128), jnp.float32),
        ],
    )(q, k)


def _attention_masked(q, k, v, imp, thr):
    def body(q_ref, k_ref, v_ref, imp_ref, thr_ref, o_ref):
        t = thr_ref[0, 0]
        keep = imp_ref[0:1, :] >= t
        outs = []
        for h in range(HEADS):
            s = lax.dot_general(_head_slice(q_ref, h), _head_slice(k_ref, h),
                                (((1,), (1,)), ((), ())),
                                preferred_element_type=jnp.float32)
            s = jnp.where(keep, s, -1e30)
            m = jnp.max(s, axis=-1, keepdims=True)
            e = jnp.exp(s - m)
            w = e * (1.0 / jnp.sum(e, axis=-1, keepdims=True))
            outs.append(_dot(w, _head_slice(v_ref, h)))
        o_ref[...] = jnp.concatenate(outs, axis=1)

    return pl.pallas_call(
        body,
        grid=(SEQ // QBLK,),
        in_specs=[
            pl.BlockSpec((QBLK, DIM), lambda i: (i, 0)),
            pl.BlockSpec((SEQ, DIM), lambda i: (0, 0)),
            pl.BlockSpec((SEQ, DIM), lambda i: (0, 0)),
            pl.BlockSpec((8, SEQ), lambda i: (0, 0)),
            pl.BlockSpec((8, 128), lambda i: (0, 0)),
        ],
        out_specs=pl.BlockSpec((QBLK, DIM), lambda i: (i, 0)),
        out_shape=jax.ShapeDtypeStruct((SEQ, DIM), jnp.float32),
    )(q, k, v, imp, thr)


def _attention_compressed(q, kc, vc, keep):
    def body(q_ref, kc_ref, vc_ref, o_ref):
        col = lax.broadcasted_iota(jnp.int32, (1, PAD), 1)
        live = col < keep
        outs = []
        for h in range(HEADS):
            s = lax.dot_general(_head_slice(q_ref, h), _head_slice(kc_ref, h),
                                (((1,), (1,)), ((), ())),
                                preferred_element_type=jnp.float32)
            s = jnp.where(live, s, -1e30)
            m = jnp.max(s, axis=-1, keepdims=True)
            e = jnp.exp(s - m)
            w = e * (1.0 / jnp.sum(e, axis=-1, keepdims=True))
            outs.append(_dot(w, _head_slice(vc_ref, h)))
        o_ref[...] = jnp.concatenate(outs, axis=1)

    return pl.pallas_call(
        body,
        grid=(SEQ // QBLK,),
        in_specs=[
            pl.BlockSpec((QBLK, DIM), lambda i: (i, 0)),
            pl.BlockSpec((PAD, DIM), lambda i: (0, 0)),
            pl.BlockSpec((PAD, DIM), lambda i: (0, 0)),
        ],
        out_specs=pl.BlockSpec((QBLK, DIM), lambda i: (i, 0)),
        out_shape=jax.ShapeDtypeStruct((SEQ, DIM), jnp.float32),
    )(q, kc, vc)


def _post_attn(x, attn, ow, ob, g, b, w1, b1, w2, b2):
    def body(x_ref, a_ref, ow_ref, ob_ref, g_ref, b_ref, w1_ref, b1_ref,
             w2_ref, b2_ref, o_ref):
        y = x_ref[...] + _dot(a_ref[...], ow_ref[...]) + ob_ref[...]
        h = _layernorm(y, g_ref[...], b_ref[...])
        f = _gelu(_dot(h, w1_ref[...]) + b1_ref[...])
        o_ref[...] = y + _dot(f, w2_ref[...]) + b2_ref[...]

    return pl.pallas_call(
        body,
        grid=(SEQ // RBLK,),
        in_specs=[
            pl.BlockSpec((RBLK, DIM), lambda i: (i, 0)),
            pl.BlockSpec((RBLK, DIM), lambda i: (i, 0)),
            pl.BlockSpec((DIM, DIM), lambda i: (0, 0)),
            pl.BlockSpec((1, DIM), lambda i: (0, 0)),
            pl.BlockSpec((1, DIM), lambda i: (0, 0)),
            pl.BlockSpec((1, DIM), lambda i: (0, 0)),
            pl.BlockSpec((DIM, FF), lambda i: (0, 0)),
            pl.BlockSpec((1, FF), lambda i: (0, 0)),
            pl.BlockSpec((FF, DIM), lambda i: (0, 0)),
            pl.BlockSpec((1, DIM), lambda i: (0, 0)),
        ],
        out_specs=pl.BlockSpec((RBLK, DIM), lambda i: (i, 0)),
        out_shape=jax.ShapeDtypeStruct((SEQ, DIM), jnp.float32),
    )(x, attn, ow, ob, g, b, w1, b1, w2, b2)


def _row(v):
    return v.reshape(1, -1)


def kernel(params, input_ids):
    ids = input_ids.reshape(-1).astype(jnp.int32)
    emb = _embed_gather(params['tok_emb'], ids)
    pos = params['pos_emb'][:SEQ]
    x = _in_proj(emb, pos, params['in_w'], _row(params['in_b']))

    for li in range(LAYERS):
        p = params['layers'][li]
        ratio = SCHEDULE[li]
        q, k, v = _ln_qkv(x, _row(p['ln1_g']), _row(p['ln1_b']),
                          p['q_w'], _row(p['q_b']),
                          p['k_w'], _row(p['k_b']),
                          p['v_w'], _row(p['v_b']))
        if ratio < 1.0:
            num_keep = max(1, int(ratio * SEQ))
            imp, _ = _importance_and_threshold(q, k, num_keep)
            posmap = _posmap(imp[0].reshape(16, 128), num_keep)
            kc, vc = _build_gather_fn(k, v, posmap.reshape(SEQ))
            attn = _attention_compressed(q, kc, vc, num_keep)
        else:
            attn = _attention_full(q, k, v)
        x = _post_attn(x, attn, p['out_w'], _row(p['out_b']),
                       _row(p['ln2_g']), _row(p['ln2_b']),
                       p['ff1_w'], _row(p['ff1_b']),
                       p['ff2_w'], _row(p['ff2_b']))

    return x.reshape(1, SEQ, DIM)
